# Initial kernel scaffold; baseline (speedup 1.0000x reference)
import jax
import jax.numpy as jnp
from jax.experimental import pallas as pl


def kernel(x, table):
    raise NotImplementedError("write your pallas kernel here")


# speedup vs baseline: 4.8380x; 4.8380x over previous
import functools

import jax
import jax.numpy as jnp
from jax import lax
from jax.experimental import pallas as pl
from jax.experimental.pallas import tpu as pltpu
from jax.experimental.pallas import tpu_sc as plsc

OUT_DIM = 64


@functools.lru_cache(maxsize=None)
def _make_gather(B, C):
    info = plsc.get_sparse_core_info()
    NC, NS = info.num_cores, info.num_subcores
    NW = NC * NS
    b_per_w = B // NW
    n_chunks = b_per_w // C
    assert b_per_w % C == 0 and B % NW == 0

    mesh = plsc.VectorSubcoreMesh(core_axis_name="c", subcore_axis_name="s")

    @functools.partial(
        pl.kernel,
        mesh=mesh,
        out_type=jax.ShapeDtypeStruct((B, OUT_DIM), jnp.float32),
        scratch_types=[
            pltpu.VMEM((C,), jnp.int32),
            pltpu.VMEM((C, OUT_DIM), jnp.float32),
            pltpu.SemaphoreType.DMA,
        ],
        compiler_params=pltpu.CompilerParams(use_tc_tiling_on_sc=False),
    )
    def k(idx_hbm, table_hbm, out_hbm, idx_v, rows_v, sem):
        wid = lax.axis_index("s") * NC + lax.axis_index("c")
        base = wid * b_per_w

        def body(g, carry):
            off = base + g * C
            pltpu.sync_copy(idx_hbm.at[pl.ds(off, C)], idx_v)
            pltpu.async_copy(table_hbm.at[idx_v], rows_v, sem).wait()
            pltpu.sync_copy(rows_v, out_hbm.at[pl.ds(off, C)])
            return carry

        lax.fori_loop(0, n_chunks, body, 0)

    return k


def kernel(x, table):
    Bt, T = x.shape
    B = Bt * T
    idx = x.reshape(B).astype(jnp.int32)
    out = _make_gather(B, 640)(idx, table)
    return out.reshape(Bt, T, OUT_DIM)


# speedup vs baseline: 5.1667x; 1.0679x over previous
import functools

import jax
import jax.numpy as jnp
from jax import lax
from jax.experimental import pallas as pl
from jax.experimental.pallas import tpu as pltpu
from jax.experimental.pallas import tpu_sc as plsc

OUT_DIM = 64


@functools.lru_cache(maxsize=None)
def _make_gather(B, C):
    info = plsc.get_sparse_core_info()
    NC, NS = info.num_cores, info.num_subcores
    NW = NC * NS
    b_per_w = B // NW
    n_chunks = b_per_w // C
    assert b_per_w % C == 0 and B % NW == 0
    assert n_chunks % 2 == 0 and n_chunks >= 6

    mesh = plsc.VectorSubcoreMesh(core_axis_name="c", subcore_axis_name="s")

    @functools.partial(
        pl.kernel,
        mesh=mesh,
        out_type=jax.ShapeDtypeStruct((B, OUT_DIM), jnp.float32),
        scratch_types=[
            pltpu.VMEM((2, C), jnp.int32),
            pltpu.VMEM((2, C, OUT_DIM), jnp.float32),
            pltpu.SemaphoreType.DMA,
            pltpu.SemaphoreType.DMA,
            pltpu.SemaphoreType.DMA,
            pltpu.SemaphoreType.DMA,
        ],
        compiler_params=pltpu.CompilerParams(use_tc_tiling_on_sc=False),
    )
    def k(idx_hbm, table_hbm, out_hbm, idx_v, rows_v, g0, g1, o0, o1):
        wid = lax.axis_index("s") * NC + lax.axis_index("c")
        base = wid * b_per_w
        gsem = [g0, g1]
        osem = [o0, o1]

        def load_idx(g, s):
            pltpu.sync_copy(idx_hbm.at[pl.ds(base + g * C, C)], idx_v.at[s])

        def start_gather(g, s):
            pltpu.async_copy(table_hbm.at[idx_v.at[s]], rows_v.at[s], gsem[s])

        def wait_gather(s):
            pltpu.make_async_copy(
                table_hbm.at[idx_v.at[s]], rows_v.at[s], gsem[s]
            ).wait()

        def start_store(g, s):
            pltpu.async_copy(
                rows_v.at[s], out_hbm.at[pl.ds(base + g * C, C)], osem[s]
            )

        def wait_store(g, s):
            pltpu.make_async_copy(
                rows_v.at[s], out_hbm.at[pl.ds(base + g * C, C)], osem[s]
            ).wait()

        def step(g, s, first=False, last=False):
            sn = 1 - s
            if not last:
                load_idx(g + 1, sn)
                if not first:
                    wait_store(g - 1, sn)
                start_gather(g + 1, sn)
            wait_gather(s)
            start_store(g, s)

        load_idx(0, 0)
        start_gather(0, 0)
        step(0, 0, first=True)
        step(1, 1)

        def body(t, carry):
            g = 2 * t
            step(g, 0)
            step(g + 1, 1)
            return carry

        lax.fori_loop(1, n_chunks // 2 - 1, body, 0)

        g_last = n_chunks - 2
        step(g_last, 0)
        step(g_last + 1, 1, last=True)
        wait_store(g_last, 0)
        wait_store(g_last + 1, 1)

    return k


def kernel(x, table):
    Bt, T = x.shape
    B = Bt * T
    idx = x.reshape(B).astype(jnp.int32)
    out = _make_gather(B, 800)(idx, table)
    return out.reshape(Bt, T, OUT_DIM)


# speedup vs baseline: 5.1691x; 1.0005x over previous
import functools

import jax
import jax.numpy as jnp
from jax import lax
from jax.experimental import pallas as pl
from jax.experimental.pallas import tpu as pltpu
from jax.experimental.pallas import tpu_sc as plsc

OUT_DIM = 64


@functools.lru_cache(maxsize=None)
def _make_gather(NB, T, KB):
    info = plsc.get_sparse_core_info()
    NC, NS = info.num_cores, info.num_subcores
    NW = NC * NS
    rows_per_w = NB // NW
    n_chunks = rows_per_w // KB
    assert NB % NW == 0 and rows_per_w % KB == 0
    assert n_chunks % 2 == 0 and n_chunks >= 6

    mesh = plsc.VectorSubcoreMesh(core_axis_name="c", subcore_axis_name="s")

    @functools.partial(
        pl.kernel,
        mesh=mesh,
        out_type=jax.ShapeDtypeStruct((NB, T, OUT_DIM), jnp.float32),
        scratch_types=[
            pltpu.VMEM((2, KB, T), jnp.int32),
            pltpu.VMEM((2, KB, T, OUT_DIM), jnp.float32),
            pltpu.SemaphoreType.DMA,
            pltpu.SemaphoreType.DMA,
            pltpu.SemaphoreType.DMA,
            pltpu.SemaphoreType.DMA,
        ],
        compiler_params=pltpu.CompilerParams(use_tc_tiling_on_sc=False),
    )
    def k(idx_hbm, table_hbm, out_hbm, idx_v, rows_v, g0, g1, o0, o1):
        wid = lax.axis_index("s") * NC + lax.axis_index("c")
        base = wid * rows_per_w
        gsem = [g0, g1]
        osem = [o0, o1]

        def load_idx(g, s):
            pltpu.sync_copy(idx_hbm.at[pl.ds(base + g * KB, KB)], idx_v.at[s])

        def start_gathers(g, s):
            for j in range(KB):
                pltpu.async_copy(
                    table_hbm.at[idx_v.at[s, j]], rows_v.at[s, j], gsem[s]
                )

        def wait_gathers(s):
            for j in range(KB):
                pltpu.make_async_copy(
                    table_hbm.at[idx_v.at[s, j]], rows_v.at[s, j], gsem[s]
                ).wait()

        def start_store(g, s):
            pltpu.async_copy(
                rows_v.at[s], out_hbm.at[pl.ds(base + g * KB, KB)], osem[s]
            )

        def wait_store(g, s):
            pltpu.make_async_copy(
                rows_v.at[s], out_hbm.at[pl.ds(base + g * KB, KB)], osem[s]
            ).wait()

        def step(g, s, first=False, last=False):
            sn = 1 - s
            if not last:
                load_idx(g + 1, sn)
                if not first:
                    wait_store(g - 1, sn)
                start_gathers(g + 1, sn)
            wait_gathers(s)
            start_store(g, s)

        load_idx(0, 0)
        start_gathers(0, 0)
        step(0, 0, first=True)
        step(1, 1)

        def body(t, carry):
            g = 2 * t
            step(g, 0)
            step(g + 1, 1)
            return carry

        lax.fori_loop(1, n_chunks // 2 - 1, body, 0)

        g_last = n_chunks - 2
        step(g_last, 0)
        step(g_last + 1, 1, last=True)
        wait_store(g_last, 0)
        wait_store(g_last + 1, 1)

    return k


def kernel(x, table):
    NB, T = x.shape
    idx = x.astype(jnp.int32)
    return _make_gather(NB, T, 4)(idx, table)
